# baseline (speedup 1.0000x reference)
import functools

import jax
import jax.numpy as jnp
from jax import lax
from jax.experimental import pallas as pl
from jax.experimental.pallas import tpu as pltpu

N = 10000
E = 320000
H = 128

_ROW_BLK = 1000
_EDGE_BLK = 2000

F32 = jnp.float32


def _sigmoid(v):
    return 1.0 / (1.0 + jnp.exp(-v))


def _encode_body(x_ref, h_ref, wx_ref, wh_ref, b_ref, wa_ref, ba_ref, wb_ref,
                 z_ref, a_ref, bo_ref):
    z = jnp.dot(x_ref[...], wx_ref[...], preferred_element_type=F32)
    z = z + jnp.dot(h_ref[...], wh_ref[...], preferred_element_type=F32)
    z = jnp.maximum(z + b_ref[...], 0.0)
    z_ref[...] = z
    a_ref[...] = jnp.dot(z, wa_ref[...], preferred_element_type=F32) + ba_ref[...]
    bo_ref[...] = jnp.dot(z, wb_ref[...], preferred_element_type=F32)


def _encode(x, h, wx, wh, b, wa, ba, wb):
    grid = (N // _ROW_BLK,)
    row = pl.BlockSpec((_ROW_BLK, H), lambda i: (i, 0))
    full = pl.BlockSpec((H, H), lambda i: (0, 0))
    vec = pl.BlockSpec((1, H), lambda i: (0, 0))
    return pl.pallas_call(
        _encode_body,
        grid=grid,
        in_specs=[row, row, full, full, vec, full, vec, full],
        out_specs=[row, row, row],
        out_shape=[jax.ShapeDtypeStruct((N, H), F32)] * 3,
    )(x, h, wx, wh, b, wa, ba, wb)


def _post_body(agg_ref, wa_ref, ba_ref, wb_ref, a_ref, bo_ref):
    g = agg_ref[...]
    g = jnp.where(jnp.isneginf(g), 0.0, g)
    h1 = jnp.maximum(g, 0.0)
    a_ref[...] = jnp.dot(h1, wa_ref[...], preferred_element_type=F32) + ba_ref[...]
    bo_ref[...] = jnp.dot(h1, wb_ref[...], preferred_element_type=F32)


def _post(agg, wa, ba, wb):
    grid = (N // _ROW_BLK,)
    row = pl.BlockSpec((_ROW_BLK, H), lambda i: (i, 0))
    full = pl.BlockSpec((H, H), lambda i: (0, 0))
    vec = pl.BlockSpec((1, H), lambda i: (0, 0))
    return pl.pallas_call(
        _post_body,
        grid=grid,
        in_specs=[row, full, vec, full],
        out_specs=[row, row],
        out_shape=[jax.ShapeDtypeStruct((N, H), F32)] * 2,
    )(agg, wa, ba, wb)


def _edge_mlp_body(m_ref, w2_ref, b2_ref, p_ref):
    m = jnp.maximum(m_ref[...], 0.0)
    p_ref[...] = jnp.dot(m, w2_ref[...], preferred_element_type=F32) + b2_ref[...]


def _edge_mlp(m, w2, b2):
    grid = (E // _EDGE_BLK,)
    row = pl.BlockSpec((_EDGE_BLK, H), lambda i: (i, 0))
    full = pl.BlockSpec((H, H), lambda i: (0, 0))
    vec = pl.BlockSpec((1, H), lambda i: (0, 0))
    return pl.pallas_call(
        _edge_mlp_body,
        grid=grid,
        in_specs=[row, full, vec],
        out_specs=row,
        out_shape=jax.ShapeDtypeStruct((E, H), F32),
    )(m, w2, b2)


def _decode_body(agg_ref, z_ref, whh_ref, wz_ref, db_ref, w1_ref, db1_ref,
                 whd_ref, hb_ref, wt_ref, tb_ref,
                 hh_ref, y_ref, t_ref, hsum_ref):
    i = pl.program_id(0)
    g = agg_ref[...]
    hh = jnp.where(jnp.isneginf(g), 0.0, g)
    hh_ref[...] = hh
    o = jnp.dot(hh, whh_ref[...], preferred_element_type=F32)
    o = o + jnp.dot(z_ref[...], wz_ref[...], preferred_element_type=F32)
    o = jnp.maximum(o + db_ref[...], 0.0)
    o = jnp.maximum(jnp.dot(o, w1_ref[...], preferred_element_type=F32) + db1_ref[...], 0.0)
    y_ref[...] = _sigmoid(jnp.dot(o, whd_ref[...], preferred_element_type=F32) + hb_ref[...])

    @pl.when(i == 0)
    def _():
        hsum_ref[...] = jnp.zeros_like(hsum_ref)

    hsum_ref[...] += jnp.sum(hh, axis=0, keepdims=True)

    @pl.when(i == pl.num_programs(0) - 1)
    def _():
        hbar = hsum_ref[...] * (1.0 / N)
        t_ref[...] = _sigmoid(
            jnp.dot(hbar, wt_ref[...], preferred_element_type=F32) + tb_ref[...])


def _decode(agg, z, whh, wz, db, w1, db1, whd, hb, wt, tb):
    grid = (N // _ROW_BLK,)
    row = pl.BlockSpec((_ROW_BLK, H), lambda i: (i, 0))
    full = pl.BlockSpec((H, H), lambda i: (0, 0))
    vec = pl.BlockSpec((1, H), lambda i: (0, 0))
    col = pl.BlockSpec((H, 1), lambda i: (0, 0))
    yblk = pl.BlockSpec((_ROW_BLK, 1), lambda i: (i, 0))
    one = pl.BlockSpec((1, 1), lambda i: (0, 0))
    return pl.pallas_call(
        _decode_body,
        grid=grid,
        in_specs=[row, row, full, full, vec, full, vec, col, one, col, one],
        out_specs=[row, yblk, one],
        out_shape=[jax.ShapeDtypeStruct((N, H), F32),
                   jax.ShapeDtypeStruct((N, 1), F32),
                   jax.ShapeDtypeStruct((1, 1), F32)],
        scratch_shapes=[pltpu.VMEM((1, H), F32)],
    )(agg, z, whh, wz, db, w1, db1, whd, hb, wt, tb)


def _gather_add(a, b, dst, src):
    return jnp.take(a, dst, axis=0) + jnp.take(b, src, axis=0)


def _segment_max(p, dst):
    neg = jnp.full((N, H), -jnp.inf, dtype=F32)
    return neg.at[dst].max(p)


def kernel(x, h, edge_index, enc_w, enc_b, conv0_w1, conv0_b1, conv0_w2,
           conv0_b2, conv1_w1, conv1_b1, conv1_w2, conv1_b2, dec_w, dec_b,
           dec_w1, dec_b1, head_w, head_b, term_w, term_b):
    src = edge_index[0]
    dst = edge_index[1]

    z, a0, b0 = _encode(
        x, h,
        enc_w[:, :H].T, enc_w[:, H:].T, enc_b.reshape(1, H),
        conv0_w1[:, :H].T, conv0_b1.reshape(1, H), conv0_w1[:, H:].T)

    m0 = _gather_add(a0, b0, dst, src)
    p0 = _edge_mlp(m0, conv0_w2.T, conv0_b2.reshape(1, H))
    agg0 = _segment_max(p0, dst)

    a1, b1 = _post(agg0, conv1_w1[:, :H].T, conv1_b1.reshape(1, H),
                   conv1_w1[:, H:].T)

    m1 = _gather_add(a1, b1, dst, src)
    p1 = _edge_mlp(m1, conv1_w2.T, conv1_b2.reshape(1, H))
    agg1 = _segment_max(p1, dst)

    hh, y, t = _decode(
        agg1, z,
        dec_w[:, :H].T, dec_w[:, H:].T, dec_b.reshape(1, H),
        dec_w1.T, dec_b1.reshape(1, H),
        head_w.T, head_b.reshape(1, 1),
        term_w.T, term_b.reshape(1, 1))

    return (y, t.reshape(1), hh)


# speedup vs baseline: 2.4697x; 2.4697x over previous
import functools

import jax
import jax.numpy as jnp
from jax import lax
from jax.experimental import pallas as pl
from jax.experimental.pallas import tpu as pltpu
from jax.experimental.pallas import tpu_sc as plsc

N = 10000
E = 320000
H = 128

_ROW_BLK = 1000
_EDGE_BLK = 2000

F32 = jnp.float32
I32 = jnp.int32

_NC = 2
_NS = 16
_NW = _NC * _NS
_SEG = 320
_NPAD = _NW * _SEG
_CAP = 16384
_SCAN = 2560
_PCH = 128
_NLCH = _CAP // _PCH
_NSUBT = E // _PCH
_SUBW = _NSUBT // _NW
_SUBREM = _NSUBT - _SUBW * _NW

_MESH = plsc.VectorSubcoreMesh(core_axis_name="c", subcore_axis_name="s")
_SC_PARAMS = pltpu.CompilerParams(needs_layout_passes=False)


def _wid():
    return lax.axis_index("s") * _NC + lax.axis_index("c")


def _sigmoid(v):
    return 1.0 / (1.0 + jnp.exp(-v))


def _encode_body(x_ref, h_ref, wx_ref, wh_ref, b_ref, wa_ref, ba_ref, wb_ref,
                 z_ref, a_ref, bo_ref):
    z = jnp.dot(x_ref[...], wx_ref[...], preferred_element_type=F32)
    z = z + jnp.dot(h_ref[...], wh_ref[...], preferred_element_type=F32)
    z = jnp.maximum(z + b_ref[...], 0.0)
    z_ref[...] = z
    a_ref[...] = jnp.dot(z, wa_ref[...], preferred_element_type=F32) + ba_ref[...]
    bo_ref[...] = jnp.dot(z, wb_ref[...], preferred_element_type=F32)


def _encode(x, h, wx, wh, b, wa, ba, wb):
    grid = (N // _ROW_BLK,)
    row = pl.BlockSpec((_ROW_BLK, H), lambda i: (i, 0))
    full = pl.BlockSpec((H, H), lambda i: (0, 0))
    vec = pl.BlockSpec((1, H), lambda i: (0, 0))
    return pl.pallas_call(
        _encode_body,
        grid=grid,
        in_specs=[row, row, full, full, vec, full, vec, full],
        out_specs=[row, row, row],
        out_shape=[jax.ShapeDtypeStruct((N, H), F32)] * 3,
    )(x, h, wx, wh, b, wa, ba, wb)


def _post_body(agg_ref, wa_ref, ba_ref, wb_ref, a_ref, bo_ref):
    g = agg_ref[...]
    g = jnp.where(jnp.isneginf(g), 0.0, g)
    h1 = jnp.maximum(g, 0.0)
    a_ref[...] = jnp.dot(h1, wa_ref[...], preferred_element_type=F32) + ba_ref[...]
    bo_ref[...] = jnp.dot(h1, wb_ref[...], preferred_element_type=F32)


def _post(agg, wa, ba, wb):
    grid = (N // _ROW_BLK,)
    row = pl.BlockSpec((_ROW_BLK, H), lambda i: (i, 0))
    full = pl.BlockSpec((H, H), lambda i: (0, 0))
    vec = pl.BlockSpec((1, H), lambda i: (0, 0))
    return pl.pallas_call(
        _post_body,
        grid=grid,
        in_specs=[row, full, vec, full],
        out_specs=[row, row],
        out_shape=[jax.ShapeDtypeStruct((N, H), F32)] * 2,
    )(agg, wa, ba, wb)


def _edge_mlp_body(m_ref, w2_ref, b2_ref, p_ref):
    m = jnp.maximum(m_ref[...], 0.0)
    p_ref[...] = jnp.dot(m, w2_ref[...], preferred_element_type=F32) + b2_ref[...]


def _edge_mlp(m, w2, b2):
    grid = (E // _EDGE_BLK,)
    row = pl.BlockSpec((_EDGE_BLK, H), lambda i: (i, 0))
    full = pl.BlockSpec((H, H), lambda i: (0, 0))
    vec = pl.BlockSpec((1, H), lambda i: (0, 0))
    return pl.pallas_call(
        _edge_mlp_body,
        grid=grid,
        in_specs=[row, full, vec],
        out_specs=row,
        out_shape=jax.ShapeDtypeStruct((E, H), F32),
    )(m, w2, b2)


def _decode_body(agg_ref, z_ref, whh_ref, wz_ref, db_ref, w1_ref, db1_ref,
                 whd_ref, hb_ref, wt_ref, tb_ref,
                 hh_ref, y_ref, t_ref, hsum_ref):
    i = pl.program_id(0)
    g = agg_ref[...]
    hh = jnp.where(jnp.isneginf(g), 0.0, g)
    hh_ref[...] = hh
    o = jnp.dot(hh, whh_ref[...], preferred_element_type=F32)
    o = o + jnp.dot(z_ref[...], wz_ref[...], preferred_element_type=F32)
    o = jnp.maximum(o + db_ref[...], 0.0)
    o = jnp.maximum(jnp.dot(o, w1_ref[...], preferred_element_type=F32) + db1_ref[...], 0.0)
    y_ref[...] = _sigmoid(jnp.dot(o, whd_ref[...], preferred_element_type=F32) + hb_ref[...])

    @pl.when(i == 0)
    def _():
        hsum_ref[...] = jnp.zeros_like(hsum_ref)

    hsum_ref[...] += jnp.sum(hh, axis=0, keepdims=True)

    @pl.when(i == pl.num_programs(0) - 1)
    def _():
        hbar = hsum_ref[...] * (1.0 / N)
        t_ref[...] = _sigmoid(
            jnp.dot(hbar, wt_ref[...], preferred_element_type=F32) + tb_ref[...])


def _decode(agg, z, whh, wz, db, w1, db1, whd, hb, wt, tb):
    grid = (N // _ROW_BLK,)
    row = pl.BlockSpec((_ROW_BLK, H), lambda i: (i, 0))
    full = pl.BlockSpec((H, H), lambda i: (0, 0))
    vec = pl.BlockSpec((1, H), lambda i: (0, 0))
    col = pl.BlockSpec((H, 1), lambda i: (0, 0))
    yblk = pl.BlockSpec((_ROW_BLK, 1), lambda i: (i, 0))
    one = pl.BlockSpec((1, 1), lambda i: (0, 0))
    return pl.pallas_call(
        _decode_body,
        grid=grid,
        in_specs=[row, row, full, full, vec, full, vec, col, one, col, one],
        out_specs=[row, yblk, one],
        out_shape=[jax.ShapeDtypeStruct((N, H), F32),
                   jax.ShapeDtypeStruct((N, 1), F32),
                   jax.ShapeDtypeStruct((1, 1), F32)],
        scratch_shapes=[pltpu.VMEM((1, H), F32)],
    )(agg, z, whh, wz, db, w1, db1, whd, hb, wt, tb)


def _bin_body(dst_hbm, pk_hbm, cnts_hbm, scan_v, pkl_v, cw_v):
    w = _wid()
    lo = w * _SEG
    hi = lo + _SEG
    junk = _CAP + 16

    def fill(i, _):
        pkl_v[pl.ds(i * 16, 16)] = jnp.full((16,), _SEG, I32)
        return 0
    lax.fori_loop(0, _CAP // 16, fill, 0, unroll=False)

    def chunk(c, cnt):
        pltpu.sync_copy(dst_hbm.at[pl.ds(pl.multiple_of(c * _SCAN, 128), _SCAN)],
                        scan_v)

        def grp(g, cnt):
            d = scan_v[pl.ds(g * 16, 16)]
            m = (d >= lo) & (d < hi)
            eid = lax.iota(I32, 16) + (c * _SCAN + g * 16)
            mi = m.astype(I32)
            rank = plsc.cumsum(mi) - mi
            addr = jnp.where(m, cnt + rank, junk)
            packed = lax.shift_left(eid, 9) | (d - lo)
            plsc.store_scatter(pkl_v, [addr], packed)
            npc = plsc.all_reduce_population_count(m)
            return cnt + npc[0]
        return lax.fori_loop(0, _SCAN // 16, grp, cnt, unroll=False)

    cnt = lax.fori_loop(0, E // _SCAN, chunk, jnp.int32(0), unroll=False)

    pltpu.sync_copy(pkl_v.at[pl.ds(0, _CAP)],
                    pk_hbm.at[pl.ds(pl.multiple_of(w * _CAP, 128), _CAP)])
    for g in range(8):
        cw_v[pl.ds(g * 16, 16)] = jnp.full((16,), 0, I32) + cnt
    pltpu.sync_copy(cw_v, cnts_hbm.at[pl.ds(pl.multiple_of(w * 128, 128), 128)])


def _sc_bin(dst):
    kfn = pl.kernel(
        _bin_body,
        mesh=_MESH,
        out_type=[jax.ShapeDtypeStruct((_NW * _CAP,), I32),
                  jax.ShapeDtypeStruct((_NW * 128,), I32)],
        compiler_params=_SC_PARAMS,
        scratch_types=[pltpu.VMEM((_SCAN,), I32),
                       pltpu.VMEM((_CAP + 32,), I32),
                       pltpu.VMEM((128,), I32)],
    )
    return kfn(dst)


def _ga_body(a_hbm, b_hbm, di_hbm, si_hbm, m_hbm,
             di_v, si_v, ba0_v, bb0_v, ba1_v, bb1_v, sem0, sem1):
    w = _wid()
    base = _SUBW * w + jnp.minimum(w, _SUBREM)
    nsub = _SUBW + (w < _SUBREM).astype(I32)
    base8 = pl.multiple_of((base // 8) * 8, 8)
    off = base - base8
    pltpu.sync_copy(di_hbm.at[pl.ds(base8, _SUBW + 10)], di_v)
    pltpu.sync_copy(si_hbm.at[pl.ds(base8, _SUBW + 10)], si_v)

    def fire(c, ba, bb, sem):
        pltpu.async_copy(a_hbm.at[di_v.at[off + c]], ba, sem)
        pltpu.async_copy(b_hbm.at[si_v.at[off + c]], bb, sem)

    def drain(c, ba, bb, sem):
        pltpu.make_async_copy(a_hbm.at[di_v.at[off + c]], ba, sem).wait()
        pltpu.make_async_copy(b_hbm.at[si_v.at[off + c]], bb, sem).wait()

    def process(c, ba, bb):
        def row(e, _):
            for j in range(8):
                s = pl.ds(j * 16, 16)
                ba[e, s] = ba[e, s] + bb[e, s]
            return 0
        lax.fori_loop(0, _PCH, row, 0, unroll=False)
        pltpu.sync_copy(
            ba, m_hbm.at[pl.ds(pl.multiple_of((base + c) * _PCH, 128), _PCH)])

    fire(0, ba0_v, bb0_v, sem0)

    def pair(i, _):
        c0 = 2 * i
        fire(c0 + 1, ba1_v, bb1_v, sem1)
        drain(c0, ba0_v, bb0_v, sem0)
        process(c0, ba0_v, bb0_v)

        @pl.when(c0 + 2 < nsub)
        def _():
            fire(c0 + 2, ba0_v, bb0_v, sem0)
        drain(c0 + 1, ba1_v, bb1_v, sem1)
        process(c0 + 1, ba1_v, bb1_v)
        return 0
    lax.fori_loop(0, _SUBW // 2, pair, 0, unroll=False)

    @pl.when(nsub > _SUBW)
    def _():
        drain(_SUBW, ba0_v, bb0_v, sem0)
        process(_SUBW, ba0_v, bb0_v)


def _sc_gather_add(a, b, di, si):
    kfn = pl.kernel(
        _ga_body,
        mesh=_MESH,
        out_type=jax.ShapeDtypeStruct((E, H), F32),
        compiler_params=_SC_PARAMS,
        scratch_types=[pltpu.VMEM((_SUBW + 10, _PCH), I32),
                       pltpu.VMEM((_SUBW + 10, _PCH), I32),
                       pltpu.VMEM((_PCH, H), F32),
                       pltpu.VMEM((_PCH, H), F32),
                       pltpu.VMEM((_PCH, H), F32),
                       pltpu.VMEM((_PCH, H), F32),
                       pltpu.SemaphoreType.DMA,
                       pltpu.SemaphoreType.DMA],
    )
    return kfn(a, b, di, si)


def _segmax_body(p_hbm, pk_hbm, cnts_hbm, agg_hbm,
                 pkl_v, idx0_v, idx1_v, pb0_v, pb1_v, acc_v, cw_v,
                 sem0, sem1):
    w = _wid()
    lo = w * _SEG
    neg = jnp.full((16,), -jnp.inf, F32)

    def initr(r, _):
        for j in range(8):
            acc_v[r, pl.ds(j * 16, 16)] = neg
        return 0
    lax.fori_loop(0, _SEG + 1, initr, 0, unroll=False)

    pltpu.sync_copy(pk_hbm.at[pl.ds(pl.multiple_of(w * _CAP, 128), _CAP)], pkl_v)
    pltpu.sync_copy(cnts_hbm.at[pl.ds(pl.multiple_of(w * 128, 128), 128)], cw_v)
    cnt = cw_v[pl.ds(0, 16)][0]
    npair = (cnt + 2 * _PCH - 1) // (2 * _PCH)

    def fire(c, idx, pb, sem):
        for g in range(_PCH // 16):
            s = pl.ds(g * 16, 16)
            idx[s] = lax.shift_right_logical(pkl_v[pl.ds(c * _PCH + g * 16, 16)], 9)
        pltpu.async_copy(p_hbm.at[idx], pb, sem)

    def drain(c, idx, pb, sem):
        pltpu.make_async_copy(p_hbm.at[idx], pb, sem).wait()

    def process(c, pb):
        def grp(g, _):
            dv = pkl_v[pl.ds(c * _PCH + g * 16, 16)] & 511
            e0 = g * 16
            for k in range(16):
                d = dv[k]
                for j in range(8):
                    s = pl.ds(j * 16, 16)
                    acc_v[d, s] = jnp.maximum(acc_v[d, s], pb[e0 + k, s])
            return 0
        lax.fori_loop(0, _PCH // 16, grp, 0, unroll=False)

    @pl.when(npair > 0)
    def _():
        fire(0, idx0_v, pb0_v, sem0)

        def pair(i, _):
            c0 = 2 * i
            fire(c0 + 1, idx1_v, pb1_v, sem1)
            drain(c0, idx0_v, pb0_v, sem0)
            process(c0, pb0_v)

            @pl.when(c0 + 2 < 2 * npair)
            def _():
                fire(c0 + 2, idx0_v, pb0_v, sem0)
            drain(c0 + 1, idx1_v, pb1_v, sem1)
            process(c0 + 1, pb1_v)
            return 0
        lax.fori_loop(0, npair, pair, 0, unroll=False)

    pltpu.sync_copy(acc_v.at[pl.ds(0, _SEG)],
                    agg_hbm.at[pl.ds(pl.multiple_of(lo, 8), _SEG)])


def _sc_segment_max(p, pk2, cnts):
    kfn = pl.kernel(
        _segmax_body,
        mesh=_MESH,
        out_type=jax.ShapeDtypeStruct((_NPAD, H), F32),
        compiler_params=_SC_PARAMS,
        scratch_types=[pltpu.VMEM((_CAP,), I32),
                       pltpu.VMEM((_PCH,), I32),
                       pltpu.VMEM((_PCH,), I32),
                       pltpu.VMEM((_PCH, H), F32),
                       pltpu.VMEM((_PCH, H), F32),
                       pltpu.VMEM((_SEG + 1, H), F32),
                       pltpu.VMEM((128,), I32),
                       pltpu.SemaphoreType.DMA,
                       pltpu.SemaphoreType.DMA],
    )
    return kfn(p, pk2, cnts)


def kernel(x, h, edge_index, enc_w, enc_b, conv0_w1, conv0_b1, conv0_w2,
           conv0_b2, conv1_w1, conv1_b1, conv1_w2, conv1_b2, dec_w, dec_b,
           dec_w1, dec_b1, head_w, head_b, term_w, term_b):
    src = edge_index[0]
    dst = edge_index[1]
    di = jnp.pad(dst.reshape(_NSUBT, _PCH), ((0, 8), (0, 0)))
    si = jnp.pad(src.reshape(_NSUBT, _PCH), ((0, 8), (0, 0)))

    pk, cnts = _sc_bin(dst)

    z, a0, b0 = _encode(
        x, h,
        enc_w[:, :H].T, enc_w[:, H:].T, enc_b.reshape(1, H),
        conv0_w1[:, :H].T, conv0_b1.reshape(1, H), conv0_w1[:, H:].T)

    m0 = _sc_gather_add(a0, b0, di, si)
    p0 = _edge_mlp(m0, conv0_w2.T, conv0_b2.reshape(1, H))
    agg0 = _sc_segment_max(p0, pk, cnts)

    a1, b1 = _post(agg0, conv1_w1[:, :H].T, conv1_b1.reshape(1, H),
                   conv1_w1[:, H:].T)

    m1 = _sc_gather_add(a1, b1, di, si)
    p1 = _edge_mlp(m1, conv1_w2.T, conv1_b2.reshape(1, H))
    agg1 = _sc_segment_max(p1, pk, cnts)

    hh, y, t = _decode(
        agg1, z,
        dec_w[:, :H].T, dec_w[:, H:].T, dec_b.reshape(1, H),
        dec_w1.T, dec_b1.reshape(1, H),
        head_w.T, head_b.reshape(1, 1),
        term_w.T, term_b.reshape(1, 1))

    return (y, t.reshape(1), hh)
